# Initial kernel scaffold; baseline (speedup 1.0000x reference)
import jax
import jax.numpy as jnp
from jax.experimental import pallas as pl


def kernel(x, target):
    raise NotImplementedError("write your pallas kernel here")


# speedup vs baseline: 2.5322x; 2.5322x over previous
import functools
import math

import jax
import jax.numpy as jnp
from jax import lax
from jax.experimental import pallas as pl
from jax.experimental.pallas import tpu as pltpu
from jax.experimental.pallas import tpu_sc as plsc

_SMOOTH = 0.1
_V = 32000
_B = 4096
_OFF = _SMOOTH / (_V - 2)
_ON = 1.0 - _SMOOTH
_C0 = _SMOOTH * math.log(_OFF) + _ON * math.log(_ON)

_NC = 2
_NS = 16
_L = 16
_NW = _NC * _NS
_RPW = _B // _NW

_BR = 256
_BC = 6400
_NR = _B // _BR
_NCB = _V // _BC


def _sc_gather_body(xflat_hbm, tgt_hbm, out_hbm, tgt_v, idx_v, g_v, part_v, sem):
    wid = lax.axis_index("s") * _NC + lax.axis_index("c")
    base = wid * _RPW
    pltpu.sync_copy(tgt_hbm.at[pl.ds(base, _RPW)], tgt_v)
    for k in range(_RPW // _L):
        t16 = tgt_v[pl.ds(k * _L, _L)]
        row16 = (base + k * _L) + lax.iota(jnp.int32, _L)
        idx_v[pl.ds(k * _L, _L)] = row16 * _V + t16
    pltpu.async_copy(xflat_hbm.at[idx_v], g_v, sem).wait()
    acc = jnp.zeros((_L,), jnp.float32)
    for k in range(_RPW // _L):
        t16 = tgt_v[pl.ds(k * _L, _L)]
        g16 = g_v[pl.ds(k * _L, _L)]
        acc = acc + jnp.where(t16 != 0, g16, 0.0)
    part_v[...] = acc
    pltpu.sync_copy(part_v, out_hbm.at[wid])


def _sc_gather(xflat, target):
    mesh = plsc.VectorSubcoreMesh(
        core_axis_name="c", subcore_axis_name="s", num_cores=_NC, num_subcores=_NS
    )
    return pl.kernel(
        _sc_gather_body,
        out_type=jax.ShapeDtypeStruct((_NW, _L), jnp.float32),
        mesh=mesh,
        scratch_types=[
            pltpu.VMEM((_RPW,), jnp.int32),
            pltpu.VMEM((_RPW,), jnp.int32),
            pltpu.VMEM((_RPW,), jnp.float32),
            pltpu.VMEM((_L,), jnp.float32),
            pltpu.SemaphoreType.DMA,
        ],
    )(xflat, target)


def _tc_body(x_ref, tgt_ref, scp_ref, out_ref):
    i = pl.program_id(0)
    j = pl.program_id(1)

    @pl.when((i == 0) & (j == 0))
    def _init():
        out_ref[0, 0] = (_OFF - _ON) * jnp.sum(scp_ref[...])

    tgt = tgt_ref[0, 0, :]
    valid = tgt != 0
    xb = x_ref[...]
    row_sums = jnp.sum(xb, axis=1)
    out_ref[0, 0] += -_OFF * jnp.sum(jnp.where(valid, row_sums, 0.0))

    @pl.when(j == 0)
    def _col0_and_const():
        validf = jnp.where(valid, 1.0, 0.0)
        col0 = xb[:, 0]
        out_ref[0, 0] += _OFF * jnp.sum(jnp.where(valid, col0, 0.0)) + _C0 * jnp.sum(
            validf
        )


def _tc_reduce(x, tgt3d, scp):
    return pl.pallas_call(
        _tc_body,
        grid=(_NR, _NCB),
        in_specs=[
            pl.BlockSpec((_BR, _BC), lambda i, j: (i, j)),
            pl.BlockSpec((1, 1, _BR), lambda i, j: (i, 0, 0)),
            pl.BlockSpec((4, 128), lambda i, j: (0, 0)),
        ],
        out_specs=pl.BlockSpec((1, 1), lambda i, j: (0, 0), memory_space=pltpu.SMEM),
        out_shape=jax.ShapeDtypeStruct((1, 1), jnp.float32),
        compiler_params=pltpu.CompilerParams(
            dimension_semantics=("arbitrary", "arbitrary")
        ),
    )(x, tgt3d, scp)


@jax.jit
def kernel(x, target):
    target = target.astype(jnp.int32)
    xflat = jnp.reshape(x, (_B * _V,))
    scp = _sc_gather(xflat, target)
    tgt3d = jnp.reshape(target, (_NR, 1, _BR))
    out = _tc_reduce(x, tgt3d, jnp.reshape(scp, (4, 128)))
    return out[0, 0]


# speedup vs baseline: 2.5517x; 1.0077x over previous
import functools
import math

import jax
import jax.numpy as jnp
from jax import lax
from jax.experimental import pallas as pl
from jax.experimental.pallas import tpu as pltpu
from jax.experimental.pallas import tpu_sc as plsc

_SMOOTH = 0.1
_V = 32000
_B = 4096
_OFF = _SMOOTH / (_V - 2)
_ON = 1.0 - _SMOOTH
_C0 = _SMOOTH * math.log(_OFF) + _ON * math.log(_ON)

_NC = 2
_NS = 16
_L = 16
_NW = _NC * _NS
_RPW = _B // _NW

_BR = 512
_BC = 6400
_NR = _B // _BR
_NCB = _V // _BC


def _sc_gather_body(xflat_hbm, tgt_hbm, out_hbm, tgt_v, idx_v, g_v, part_v, sem):
    wid = lax.axis_index("s") * _NC + lax.axis_index("c")
    base = wid * _RPW
    pltpu.sync_copy(tgt_hbm.at[pl.ds(base, _RPW)], tgt_v)
    for k in range(_RPW // _L):
        t16 = tgt_v[pl.ds(k * _L, _L)]
        row16 = (base + k * _L) + lax.iota(jnp.int32, _L)
        idx_v[pl.ds(k * _L, _L)] = row16 * _V + t16
    pltpu.async_copy(xflat_hbm.at[idx_v], g_v, sem).wait()
    acc = jnp.zeros((_L,), jnp.float32)
    for k in range(_RPW // _L):
        t16 = tgt_v[pl.ds(k * _L, _L)]
        g16 = g_v[pl.ds(k * _L, _L)]
        acc = acc + jnp.where(t16 != 0, g16, 0.0)
    part_v[...] = acc
    pltpu.sync_copy(part_v, out_hbm.at[wid])


def _sc_gather(xflat, target):
    mesh = plsc.VectorSubcoreMesh(
        core_axis_name="c", subcore_axis_name="s", num_cores=_NC, num_subcores=_NS
    )
    return pl.kernel(
        _sc_gather_body,
        out_type=jax.ShapeDtypeStruct((_NW, _L), jnp.float32),
        mesh=mesh,
        scratch_types=[
            pltpu.VMEM((_RPW,), jnp.int32),
            pltpu.VMEM((_RPW,), jnp.int32),
            pltpu.VMEM((_RPW,), jnp.float32),
            pltpu.VMEM((_L,), jnp.float32),
            pltpu.SemaphoreType.DMA,
        ],
    )(xflat, target)


def _tc_body(x_ref, tgt_ref, scp_ref, out_ref):
    i = pl.program_id(0)
    j = pl.program_id(1)

    @pl.when((i == 0) & (j == 0))
    def _init():
        out_ref[0, 0] = (_OFF - _ON) * jnp.sum(scp_ref[...])

    tgt = tgt_ref[0, 0, :]
    valid = tgt != 0
    xb = x_ref[...]
    row_sums = jnp.sum(xb, axis=1)
    out_ref[0, 0] += -_OFF * jnp.sum(jnp.where(valid, row_sums, 0.0))

    @pl.when(j == 0)
    def _col0_and_const():
        validf = jnp.where(valid, 1.0, 0.0)
        col0 = xb[:, 0]
        out_ref[0, 0] += _OFF * jnp.sum(jnp.where(valid, col0, 0.0)) + _C0 * jnp.sum(
            validf
        )


def _tc_reduce(x, tgt3d, scp):
    return pl.pallas_call(
        _tc_body,
        grid=(_NR, _NCB),
        in_specs=[
            pl.BlockSpec((_BR, _BC), lambda i, j: (i, j)),
            pl.BlockSpec((1, 1, _BR), lambda i, j: (i, 0, 0)),
            pl.BlockSpec((4, 128), lambda i, j: (0, 0)),
        ],
        out_specs=pl.BlockSpec((1, 1), lambda i, j: (0, 0), memory_space=pltpu.SMEM),
        out_shape=jax.ShapeDtypeStruct((1, 1), jnp.float32),
        compiler_params=pltpu.CompilerParams(
            dimension_semantics=("arbitrary", "arbitrary")
        ),
    )(x, tgt3d, scp)


@jax.jit
def kernel(x, target):
    target = target.astype(jnp.int32)
    xflat = jnp.reshape(x, (_B * _V,))
    scp = _sc_gather(xflat, target)
    tgt3d = jnp.reshape(target, (_NR, 1, _BR))
    out = _tc_reduce(x, tgt3d, jnp.reshape(scp, (4, 128)))
    return out[0, 0]
